# baseline (speedup 1.0000x reference)
import functools

import jax
import jax.numpy as jnp
from jax import lax
from jax.experimental import pallas as pl
from jax.experimental.pallas import tpu as pltpu
from jax.experimental.pallas import tpu_sc as plsc


def kernel(z, mask):
    B, S, D = z.shape
    rows = B * S
    zf = z.reshape(rows * D)
    info = plsc.get_sparse_core_info()
    NC, NS, L = info.num_cores, info.num_subcores, info.num_lanes
    NW = NC * NS
    rows_per_w = rows // NW
    CH = 16
    n_chunks = rows_per_w // CH
    chunk_words = CH * D
    mesh = plsc.VectorSubcoreMesh(core_axis_name="c", subcore_axis_name="s")

    @functools.partial(
        pl.kernel,
        mesh=mesh,
        out_type=jax.ShapeDtypeStruct((rows * D,), jnp.float32),
        scratch_types=[
            pltpu.VMEM((D,), jnp.float32),
            pltpu.VMEM((chunk_words,), jnp.float32),
        ],
    )
    def k(z_hbm, mask_hbm, out_hbm, mask_v, buf_v):
        wid = lax.axis_index("s") * NC + lax.axis_index("c")
        base = wid * rows_per_w * D
        pltpu.sync_copy(mask_hbm, mask_v)

        def chunk(c, carry):
            off = base + c * chunk_words
            pltpu.sync_copy(z_hbm.at[pl.ds(off, chunk_words)], buf_v)

            def j_body(j, inner):
                m = mask_v[pl.ds(j * L, L)]

                def r_body(r, inner2):
                    o = r * D + j * L
                    buf_v[pl.ds(o, L)] = buf_v[pl.ds(o, L)] * m
                    return inner2

                return lax.fori_loop(0, CH, r_body, inner)

            lax.fori_loop(0, D // L, j_body, 0)
            pltpu.sync_copy(buf_v, out_hbm.at[pl.ds(off, chunk_words)])
            return carry

        lax.fori_loop(0, n_chunks, chunk, 0)

    out = k(zf, mask)
    return out.reshape(B, S, D)


# speedup vs baseline: 1.1718x; 1.1718x over previous
import functools

import jax
import jax.numpy as jnp
from jax import lax
from jax.experimental import pallas as pl
from jax.experimental.pallas import tpu as pltpu
from jax.experimental.pallas import tpu_sc as plsc


def kernel(z, mask):
    B, S, D = z.shape
    rows = B * S
    zf = z.reshape(rows * D)
    info = plsc.get_sparse_core_info()
    NC, NS, L = info.num_cores, info.num_subcores, info.num_lanes
    NW = NC * NS
    rows_per_w = rows // NW
    CH = 4
    n_chunks = rows_per_w // CH
    CW = CH * D
    mesh = plsc.VectorSubcoreMesh(core_axis_name="c", subcore_axis_name="s")

    @functools.partial(
        pl.kernel,
        mesh=mesh,
        out_type=jax.ShapeDtypeStruct((rows * D,), jnp.float32),
        scratch_types=[
            pltpu.VMEM((D,), jnp.float32),
            pltpu.VMEM((CW,), jnp.float32),
            pltpu.VMEM((CW,), jnp.float32),
            pltpu.VMEM((CW,), jnp.float32),
            pltpu.VMEM((CW,), jnp.float32),
            pltpu.SemaphoreType.DMA,
            pltpu.SemaphoreType.DMA,
            pltpu.SemaphoreType.DMA,
            pltpu.SemaphoreType.DMA,
        ],
    )
    def k(z_hbm, mask_hbm, out_hbm, mask_v, in0, in1, ou0, ou1,
          si0, si1, so0, so1):
        INB, OUB, SI, SO = [in0, in1], [ou0, ou1], [si0, si1], [so0, so1]
        wid = lax.axis_index("s") * NC + lax.axis_index("c")
        base = wid * rows_per_w * D
        pltpu.sync_copy(mask_hbm, mask_v)

        def start_in(b, j):
            pltpu.async_copy(z_hbm.at[pl.ds(base + j * CW, CW)], INB[b], SI[b])

        def wait_in(b, j):
            pltpu.make_async_copy(
                z_hbm.at[pl.ds(base + j * CW, CW)], INB[b], SI[b]).wait()

        def start_out(b, j):
            pltpu.async_copy(OUB[b], out_hbm.at[pl.ds(base + j * CW, CW)], SO[b])

        def wait_out(b, j):
            pltpu.make_async_copy(
                OUB[b], out_hbm.at[pl.ds(base + j * CW, CW)], SO[b]).wait()

        def compute(b):
            @plsc.parallel_loop(0, D // L, unroll=4)
            def _(j2):
                m = mask_v[pl.ds(j2 * L, L)]
                for r in range(CH):
                    o = r * D + j2 * L
                    OUB[b][pl.ds(o, L)] = INB[b][pl.ds(o, L)] * m

        start_in(0, 0)
        start_in(1, 1)
        for b in range(2):
            wait_in(b, b)
            compute(b)
            start_out(b, b)
            start_in(b, b + 2)

        @pl.loop(2, n_chunks - 2, step=2)
        def _(c):
            for b in range(2):
                j = c + b
                wait_in(b, j)
                wait_out(b, j - 2)
                compute(b)
                start_out(b, j)
                start_in(b, j + 2)

        for b in range(2):
            j = n_chunks - 2 + b
            wait_in(b, j)
            wait_out(b, j - 2)
            compute(b)
            start_out(b, j)
        for b in range(2):
            wait_out(b, n_chunks - 2 + b)

    out = k(zf, mask)
    return out.reshape(B, S, D)


# speedup vs baseline: 4.8107x; 4.1054x over previous
import jax
import jax.numpy as jnp
from jax.experimental import pallas as pl


def _body(z_ref, mask_ref, out_ref):
    out_ref[...] = z_ref[...] * mask_ref[...]


def kernel(z, mask):
    B, S, D = z.shape
    rows = B * S
    z2 = z.reshape(rows, D)
    BR = 256
    grid = (rows // BR,)
    out = pl.pallas_call(
        _body,
        grid=grid,
        in_specs=[
            pl.BlockSpec((BR, D), lambda i: (i, 0)),
            pl.BlockSpec((1, D), lambda i: (0, 0)),
        ],
        out_specs=pl.BlockSpec((BR, D), lambda i: (i, 0)),
        out_shape=jax.ShapeDtypeStruct((rows, D), z.dtype),
    )(z2, mask.reshape(1, D))
    return out.reshape(B, S, D)


# speedup vs baseline: 4.9005x; 1.0187x over previous
import jax
import jax.numpy as jnp
from jax.experimental import pallas as pl


def _body(z_ref, mask_ref, out_ref):
    out_ref[...] = z_ref[...] * mask_ref[...]


def kernel(z, mask):
    B, S, D = z.shape
    rows = B * S
    z2 = z.reshape(rows, D)
    BR = 512
    grid = (rows // BR,)
    out = pl.pallas_call(
        _body,
        grid=grid,
        in_specs=[
            pl.BlockSpec((BR, D), lambda i: (i, 0)),
            pl.BlockSpec((1, D), lambda i: (0, 0)),
        ],
        out_specs=pl.BlockSpec((BR, D), lambda i: (i, 0)),
        out_shape=jax.ShapeDtypeStruct((rows, D), z.dtype),
    )(z2, mask.reshape(1, D))
    return out.reshape(B, S, D)
